# Initial kernel scaffold; baseline (speedup 1.0000x reference)
import jax
import jax.numpy as jnp
from jax.experimental import pallas as pl


def kernel(x):
    raise NotImplementedError("write your pallas kernel here")


# speedup vs baseline: 9.9189x; 9.9189x over previous
import jax
import jax.numpy as jnp
from jax.experimental import pallas as pl

_BISECT_ITERS = 30


def _body(x_ref, o_ref):
    x = x_ref[...]
    m = jnp.max(x, axis=-1, keepdims=True)
    lo = m - 1.0
    hi = m

    def it(_, carry):
        lo, hi = carry
        mid = 0.5 * (lo + hi)
        s = jnp.sum(jnp.maximum(x - mid, 0.0), axis=-1, keepdims=True)
        pred = s >= 1.0
        return jnp.where(pred, mid, lo), jnp.where(pred, hi, mid)

    lo, hi = jax.lax.fori_loop(0, _BISECT_ITERS, it, (lo, hi))
    mask = x > lo
    c = jnp.sum(mask.astype(x.dtype), axis=-1, keepdims=True)
    s = jnp.sum(jnp.where(mask, x, 0.0), axis=-1, keepdims=True)
    tau = (s - 1.0) / c
    o_ref[...] = jnp.maximum(x - tau, 0.0)


def kernel(x):
    n_rows, n_cols = x.shape
    block_rows = 16
    return pl.pallas_call(
        _body,
        grid=(n_rows // block_rows,),
        in_specs=[pl.BlockSpec((block_rows, n_cols), lambda i: (i, 0))],
        out_specs=pl.BlockSpec((block_rows, n_cols), lambda i: (i, 0)),
        out_shape=jax.ShapeDtypeStruct(x.shape, x.dtype),
    )(x)
